# Initial kernel scaffold; baseline (speedup 1.0000x reference)
import jax
import jax.numpy as jnp
from jax.experimental import pallas as pl


def kernel(hidden_states, router_weight, e_score_correction_bias, w1, w3, w2):
    raise NotImplementedError("write your pallas kernel here")


# speedup vs baseline: 1.5477x; 1.5477x over previous
import functools

import jax
import jax.numpy as jnp
from jax import lax
from jax.experimental import pallas as pl
from jax.experimental.pallas import tpu as pltpu

T = 2048
H = 1024
I = 512
E = 8
Z = 2
NE = E + Z
LANES = 128
TBLK = 256
NEG = -1e30


def _router_body(x_ref, rw_ref, bias_ref, cw_ref):
    x = x_ref[...]
    logits = lax.dot_general(x, rw_ref[...], (((1,), (1,)), ((), ())),
                             preferred_element_type=jnp.float32)
    col = lax.broadcasted_iota(jnp.int32, (TBLK, LANES), 1)
    valid = col < NE
    logits = jnp.where(valid, logits, NEG)
    m = jnp.max(logits, axis=1, keepdims=True)
    ex = jnp.exp(logits - m)
    ex = jnp.where(valid, ex, 0.0)
    scores = ex / jnp.sum(ex, axis=1, keepdims=True)
    biased = jnp.where(valid, scores + bias_ref[...], NEG)

    m1 = jnp.max(biased, axis=1, keepdims=True)
    idx1 = jnp.min(jnp.where(biased == m1, col, LANES), axis=1, keepdims=True)
    oh1 = col == idx1
    w1 = jnp.sum(jnp.where(oh1, scores, 0.0), axis=1, keepdims=True)
    b2 = jnp.where(oh1, NEG, biased)
    m2 = jnp.max(b2, axis=1, keepdims=True)
    idx2 = jnp.min(jnp.where(b2 == m2, col, LANES), axis=1, keepdims=True)
    oh2 = col == idx2
    w2 = jnp.sum(jnp.where(oh2, scores, 0.0), axis=1, keepdims=True)

    cw = jnp.where(oh1, w1, 0.0) + jnp.where(oh2, w2, 0.0)
    zw = jnp.where(idx1 >= E, w1, 0.0) + jnp.where(idx2 >= E, w2, 0.0)
    cw = jnp.where(col < E, cw, jnp.where(col == E, zw, 0.0))
    cw_ref[...] = cw


def _expert_body(x_ref, w1_ref, w3_ref, w2_ref, cw_ref, out_ref):
    e = pl.program_id(1)
    x = x_ref[...]
    cw = cw_ref[...]
    col = lax.broadcasted_iota(jnp.int32, (TBLK, LANES), 1)
    cw_e = jnp.sum(jnp.where(col == e, cw, 0.0), axis=1, keepdims=True)

    @pl.when(e == 0)
    def _init():
        zw = jnp.sum(jnp.where(col == E, cw, 0.0), axis=1, keepdims=True)
        out_ref[...] = zw * x

    g = lax.dot_general(x, w1_ref[0], (((1,), (1,)), ((), ())),
                        preferred_element_type=jnp.float32)
    u = lax.dot_general(x, w3_ref[0], (((1,), (1,)), ((), ())),
                        preferred_element_type=jnp.float32)
    act = (g * jax.nn.sigmoid(g)) * u
    h = lax.dot_general(act, w2_ref[0], (((1,), (1,)), ((), ())),
                        preferred_element_type=jnp.float32)
    out_ref[...] += cw_e * h


@jax.jit
def kernel(hidden_states, router_weight, e_score_correction_bias, w1, w3, w2):
    x = hidden_states.astype(jnp.float32)
    rw = jnp.zeros((LANES, H), jnp.float32).at[:NE].set(router_weight)
    bias = jnp.full((1, LANES), NEG, jnp.float32).at[0, :NE].set(
        e_score_correction_bias)

    nt = T // TBLK
    cw = pl.pallas_call(
        _router_body,
        grid=(nt,),
        in_specs=[
            pl.BlockSpec((TBLK, H), lambda t: (t, 0)),
            pl.BlockSpec((LANES, H), lambda t: (0, 0)),
            pl.BlockSpec((1, LANES), lambda t: (0, 0)),
        ],
        out_specs=pl.BlockSpec((TBLK, LANES), lambda t: (t, 0)),
        out_shape=jax.ShapeDtypeStruct((T, LANES), jnp.float32),
    )(x, rw, bias)

    out = pl.pallas_call(
        _expert_body,
        grid=(nt, E),
        in_specs=[
            pl.BlockSpec((TBLK, H), lambda t, e: (t, 0)),
            pl.BlockSpec((1, I, H), lambda t, e: (e, 0, 0)),
            pl.BlockSpec((1, I, H), lambda t, e: (e, 0, 0)),
            pl.BlockSpec((1, H, I), lambda t, e: (e, 0, 0)),
            pl.BlockSpec((TBLK, LANES), lambda t, e: (t, 0)),
        ],
        out_specs=pl.BlockSpec((TBLK, H), lambda t, e: (t, 0)),
        out_shape=jax.ShapeDtypeStruct((T, H), jnp.float32),
        compiler_params=pltpu.CompilerParams(
            dimension_semantics=("parallel", "arbitrary")),
    )(x, w1, w3, w2, cw)
    return out


# speedup vs baseline: 1.6906x; 1.0924x over previous
import functools

import jax
import jax.numpy as jnp
from jax import lax
from jax.experimental import pallas as pl
from jax.experimental.pallas import tpu as pltpu

T = 2048
H = 1024
I = 512
E = 8
Z = 2
NE = E + Z
LANES = 128
TBLK = 256
NEG = -1e30


def _router_body(x_ref, rw_ref, bias_ref, cw_ref):
    x = x_ref[...]
    logits = lax.dot_general(x, rw_ref[...], (((1,), (1,)), ((), ())),
                             preferred_element_type=jnp.float32)
    col = lax.broadcasted_iota(jnp.int32, (TBLK, LANES), 1)
    valid = col < NE
    logits = jnp.where(valid, logits, NEG)
    m = jnp.max(logits, axis=1, keepdims=True)
    ex = jnp.exp(logits - m)
    ex = jnp.where(valid, ex, 0.0)
    scores = ex / jnp.sum(ex, axis=1, keepdims=True)
    biased = jnp.where(valid, scores + bias_ref[...], NEG)

    m1 = jnp.max(biased, axis=1, keepdims=True)
    idx1 = jnp.min(jnp.where(biased == m1, col, LANES), axis=1, keepdims=True)
    oh1 = col == idx1
    w1 = jnp.sum(jnp.where(oh1, scores, 0.0), axis=1, keepdims=True)
    b2 = jnp.where(oh1, NEG, biased)
    m2 = jnp.max(b2, axis=1, keepdims=True)
    idx2 = jnp.min(jnp.where(b2 == m2, col, LANES), axis=1, keepdims=True)
    oh2 = col == idx2
    w2 = jnp.sum(jnp.where(oh2, scores, 0.0), axis=1, keepdims=True)

    cw = jnp.where(oh1, w1, 0.0) + jnp.where(oh2, w2, 0.0)
    zw = jnp.where(idx1 >= E, w1, 0.0) + jnp.where(idx2 >= E, w2, 0.0)
    cw = jnp.where(col < E, cw, jnp.where(col == E, zw, 0.0))
    cw_ref[...] = cw


def _expert_body(x_ref, w1_ref, w3_ref, w2_ref, cw_ref, out_ref):
    e = pl.program_id(1)
    x = x_ref[...]
    cw = cw_ref[...]
    col = lax.broadcasted_iota(jnp.int32, (TBLK, LANES), 1)
    cw_e = jnp.sum(jnp.where(col == e, cw, 0.0), axis=1, keepdims=True)

    @pl.when(e == 0)
    def _init():
        zw = jnp.sum(jnp.where(col == E, cw, 0.0), axis=1, keepdims=True)
        out_ref[...] = zw * x

    xb = x.astype(jnp.bfloat16)
    g = lax.dot_general(xb, w1_ref[0], (((1,), (1,)), ((), ())),
                        preferred_element_type=jnp.float32)
    u = lax.dot_general(xb, w3_ref[0], (((1,), (1,)), ((), ())),
                        preferred_element_type=jnp.float32)
    act = ((g * jax.nn.sigmoid(g)) * u).astype(jnp.bfloat16)
    h = lax.dot_general(act, w2_ref[0], (((1,), (1,)), ((), ())),
                        preferred_element_type=jnp.float32)
    out_ref[...] += cw_e * h


@jax.jit
def kernel(hidden_states, router_weight, e_score_correction_bias, w1, w3, w2):
    x = hidden_states.astype(jnp.float32)
    rw = jnp.zeros((LANES, H), jnp.float32).at[:NE].set(router_weight)
    bias = jnp.full((1, LANES), NEG, jnp.float32).at[0, :NE].set(
        e_score_correction_bias)

    nt = T // TBLK
    cw = pl.pallas_call(
        _router_body,
        grid=(nt,),
        in_specs=[
            pl.BlockSpec((TBLK, H), lambda t: (t, 0)),
            pl.BlockSpec((LANES, H), lambda t: (0, 0)),
            pl.BlockSpec((1, LANES), lambda t: (0, 0)),
        ],
        out_specs=pl.BlockSpec((TBLK, LANES), lambda t: (t, 0)),
        out_shape=jax.ShapeDtypeStruct((T, LANES), jnp.float32),
    )(x, rw, bias)

    w1b = w1.astype(jnp.bfloat16)
    w3b = w3.astype(jnp.bfloat16)
    w2b = w2.astype(jnp.bfloat16)
    out = pl.pallas_call(
        _expert_body,
        grid=(nt, E),
        in_specs=[
            pl.BlockSpec((TBLK, H), lambda t, e: (t, 0)),
            pl.BlockSpec((1, I, H), lambda t, e: (e, 0, 0)),
            pl.BlockSpec((1, I, H), lambda t, e: (e, 0, 0)),
            pl.BlockSpec((1, H, I), lambda t, e: (e, 0, 0)),
            pl.BlockSpec((TBLK, LANES), lambda t, e: (t, 0)),
        ],
        out_specs=pl.BlockSpec((TBLK, H), lambda t, e: (t, 0)),
        out_shape=jax.ShapeDtypeStruct((T, H), jnp.float32),
        compiler_params=pltpu.CompilerParams(
            dimension_semantics=("parallel", "arbitrary")),
    )(x, w1b, w3b, w2b, cw)
    return out
